# baseline (speedup 1.0000x reference)
import functools

import jax
import jax.numpy as jnp
from jax import lax
from jax.experimental import pallas as pl
from jax.experimental.pallas import tpu as pltpu
from jax.experimental.pallas import tpu_sc as plsc

N = 10000
C = 128
E = 320000
NC, NS = 2, 16
NW = NC * NS
K = 128
CH = 79
EW = K * CH
EPAD = NW * EW
NP = CH * 128
NB = NP // 128
ZR = NP // NS
DEGW = 16

_MESH = plsc.VectorSubcoreMesh(
    core_axis_name="c", subcore_axis_name="s", num_cores=NC, num_subcores=NS)


def _sc_deg_body(src_hbm, dst_hbm, ones_hbm, zeros_hbm, cnt_hbm,
                 iv0, iv1, rows, zrow, acc):
    cid = lax.axis_index("c")
    sid = lax.axis_index("s")
    wid = sid * NC + cid
    base = wid * EW
    pltpu.sync_copy(ones_hbm, rows)
    pltpu.sync_copy(zeros_hbm, zrow)
    for j in range(5):
        blk = sid + j * NS
        @pl.when(blk < NB)
        def _():
            pltpu.sync_copy(zrow, acc.at[pl.ds(blk * 128, 128)])

    for phase, idx_hbm in ((0, dst_hbm), (1, src_hbm)):
        plsc.subcore_barrier()
        pltpu.sync_copy(idx_hbm.at[pl.ds(base, K)], iv0)

        def pair(i, carry):
            pltpu.sync_copy(idx_hbm.at[pl.ds(base + (2 * i + 1) * K, K)],
                            iv1)
            pltpu.sync_copy(rows, acc.at[iv0], add=True)
            pltpu.sync_copy(idx_hbm.at[pl.ds(base + (2 * i + 2) * K, K)],
                            iv0)
            pltpu.sync_copy(rows, acc.at[iv1], add=True)
            return carry

        lax.fori_loop(0, (CH - 1) // 2, pair, 0)
        pltpu.sync_copy(rows, acc.at[iv0], add=True)
        plsc.subcore_barrier()
        for j in range(5):
            blk = sid + j * NS
            @pl.when(blk < NB)
            def _():
                pltpu.sync_copy(
                    acc.at[pl.ds(blk * 128, 128)],
                    cnt_hbm.at[pl.ds((phase * NC + cid) * NP + blk * 128,
                                     128)])
                if phase == 0:
                    pltpu.sync_copy(zrow, acc.at[pl.ds(blk * 128, 128)])


_sc_deg = functools.partial(
    pl.kernel,
    out_type=jax.ShapeDtypeStruct((2 * NC * NP, C), jnp.float32),
    mesh=_MESH,
    scratch_types=[
        pltpu.VMEM((K,), jnp.int32),
        pltpu.VMEM((K,), jnp.int32),
        pltpu.VMEM((K, C), jnp.float32),
        pltpu.VMEM((128, C), jnp.float32),
        pltpu.VMEM_SHARED((NP, C), jnp.float32),
    ],
)(_sc_deg_body)

CHT = 2 * CH
CH0 = 139
CH1 = CHT - CH0


def _sc_gs_body(za_hbm, gi_hbm, si_hbm, zeros_hbm, out_hbm,
                giv0, giv1, siv0, siv1, rows0, rows1, acc, gsem0, gsem1):
    cid = lax.axis_index("c")
    sid = lax.axis_index("s")
    base = (sid * CHT + cid * CH0) * K
    nch = jnp.where(cid == 0, CH0, CH1)
    pltpu.sync_copy(zeros_hbm, rows0)
    for j in range(5):
        blk = sid + j * NS
        @pl.when(blk < NB)
        def _():
            pltpu.sync_copy(rows0, acc.at[pl.ds(blk * 128, 128)])
    plsc.subcore_barrier()

    pltpu.sync_copy(gi_hbm.at[pl.ds(base, K)], giv0)
    pltpu.sync_copy(si_hbm.at[pl.ds(base, K)], siv0)
    pltpu.async_copy(za_hbm.at[giv0], rows0, gsem0)

    def pair(i, carry):
        off1 = base + (2 * i + 1) * K
        pltpu.sync_copy(gi_hbm.at[pl.ds(off1, K)], giv1)
        pltpu.sync_copy(si_hbm.at[pl.ds(off1, K)], siv1)
        pltpu.async_copy(za_hbm.at[giv1], rows1, gsem1)
        pltpu.make_async_copy(za_hbm.at[giv0], rows0, gsem0).wait()
        pltpu.sync_copy(rows0, acc.at[siv0], add=True)
        off2 = base + (2 * i + 2) * K
        pltpu.sync_copy(gi_hbm.at[pl.ds(off2, K)], giv0)
        pltpu.sync_copy(si_hbm.at[pl.ds(off2, K)], siv0)
        pltpu.async_copy(za_hbm.at[giv0], rows0, gsem0)
        pltpu.make_async_copy(za_hbm.at[giv1], rows1, gsem1).wait()
        pltpu.sync_copy(rows1, acc.at[siv1], add=True)
        return carry

    lax.fori_loop(0, (nch - 1) // 2, pair, 0)
    pltpu.make_async_copy(za_hbm.at[giv0], rows0, gsem0).wait()
    pltpu.sync_copy(rows0, acc.at[siv0], add=True)
    plsc.subcore_barrier()
    for j in range(5):
        blk = sid + j * NS
        @pl.when(blk < NB)
        def _():
            pltpu.sync_copy(acc.at[pl.ds(blk * 128, 128)],
                            out_hbm.at[pl.ds(cid * NP + blk * 128, 128)])


_sc_gs = functools.partial(
    pl.kernel,
    out_type=jax.ShapeDtypeStruct((NC * NP, C), jnp.float32),
    mesh=_MESH,
    scratch_types=(
        [pltpu.VMEM((K,), jnp.int32) for _ in range(4)]
        + [pltpu.VMEM((K, C), jnp.float32) for _ in range(2)]
        + [pltpu.VMEM_SHARED((NP, C), jnp.float32)]
        + [pltpu.SemaphoreType.DMA for _ in range(2)]
    ),
)(_sc_gs_body)


_DN = (((1,), (1,)), ((), ()))
_RB = 1024
_GRID = (NP + _RB - 1) // _RB


def _tc_first_body(x_ref, w_ref, b_ref, co0, co1, ci0, ci1,
                   z1_ref, za1_ref, so_ref, si_ref):
    z1 = lax.dot_general(
        x_ref[...], w_ref[...], _DN,
        preferred_element_type=jnp.float32) + b_ref[...]
    s_o = lax.rsqrt(co0[...] + co1[...] + 1.0)
    s_i = lax.rsqrt(ci0[...] + ci1[...] + 1.0)
    z1_ref[...] = z1
    za1_ref[...] = s_o * z1
    so_ref[...] = s_o
    si_ref[...] = s_i


def _tc_first(x, w, b, co0, co1, ci0, ci1):
    col = pl.BlockSpec((_RB, 1), lambda i: (i, 0))
    mat = pl.BlockSpec((_RB, C), lambda i: (i, 0))
    return pl.pallas_call(
        _tc_first_body,
        grid=(_GRID,),
        in_specs=[mat,
                  pl.BlockSpec((C, C), lambda i: (0, 0)),
                  pl.BlockSpec((1, C), lambda i: (0, 0)),
                  col, col, col, col],
        out_specs=[mat, mat, col, col],
        out_shape=[jax.ShapeDtypeStruct((NP, C), jnp.float32),
                   jax.ShapeDtypeStruct((NP, C), jnp.float32),
                   jax.ShapeDtypeStruct((NP, 1), jnp.float32),
                   jax.ShapeDtypeStruct((NP, 1), jnp.float32)],
    )(x, w, b, co0, co1, ci0, ci1)


def _tc_mid_body(p_ref, z1_ref, so_ref, si_ref, w_ref, b_ref, z2_ref, za2_ref):
    h = jnp.maximum(
        si_ref[...] * (p_ref[0] + p_ref[1] + so_ref[...] * z1_ref[...]), 0.0)
    z2 = lax.dot_general(
        h, w_ref[...], _DN, preferred_element_type=jnp.float32) + b_ref[...]
    z2_ref[...] = z2
    za2_ref[...] = si_ref[...] * z2


def _tc_mid(p, z1, so, si, w, b):
    col = pl.BlockSpec((_RB, 1), lambda i: (i, 0))
    mat = pl.BlockSpec((_RB, C), lambda i: (i, 0))
    return pl.pallas_call(
        _tc_mid_body,
        grid=(_GRID,),
        in_specs=[pl.BlockSpec((NC, _RB, C), lambda i: (0, i, 0)),
                  mat, col, col,
                  pl.BlockSpec((C, C), lambda i: (0, 0)),
                  pl.BlockSpec((1, C), lambda i: (0, 0))],
        out_specs=[mat, mat],
        out_shape=[jax.ShapeDtypeStruct((NP, C), jnp.float32),
                   jax.ShapeDtypeStruct((NP, C), jnp.float32)],
    )(p, z1, so, si, w, b)


def _tc_final_body(q_ref, z2_ref, so_ref, si_ref, o_ref):
    o_ref[...] = so_ref[...] * (
        q_ref[0] + q_ref[1] + si_ref[...] * z2_ref[...])


def _tc_final(q, z2, so, si):
    col = pl.BlockSpec((_RB, 1), lambda i: (i, 0))
    mat = pl.BlockSpec((_RB, C), lambda i: (i, 0))
    return pl.pallas_call(
        _tc_final_body,
        grid=(_GRID,),
        in_specs=[pl.BlockSpec((NC, _RB, C), lambda i: (0, i, 0)),
                  mat, col, col],
        out_specs=mat,
        out_shape=jax.ShapeDtypeStruct((NP, C), jnp.float32),
    )(q, z2, so, si)


def _sc_testa_body(za_hbm, gi_hbm, out_hbm, giv, rows, sem):
    cid = lax.axis_index("c")
    sid = lax.axis_index("s")
    wid = sid * NC + cid
    base = wid * EW

def kernel(x, edge_index, W1, b1, W2, b2):
    src = edge_index[0].astype(jnp.int32)
    dst = edge_index[1].astype(jnp.int32)
    pad = jnp.full((EPAD - E,), N, jnp.int32)
    src_p = jnp.concatenate([src, pad])
    dst_p = jnp.concatenate([dst, pad])
    x_p = jnp.pad(x, ((0, NP - N), (0, 0)))
    ones_k = jnp.ones((K, C), jnp.float32)
    zeros_row = jnp.zeros((128, C), jnp.float32)

    cb = _sc_deg(src_p, dst_p, ones_k, zeros_row)
    z1, za1, s_out, s_in = _tc_first(
        x_p, W1, b1.reshape(1, C),
        cb[2 * NP:3 * NP, 0:1], cb[3 * NP:, 0:1],
        cb[:NP, 0:1], cb[NP:2 * NP, 0:1])
    p1 = _sc_gs(za1, src_p, dst_p, zeros_row).reshape(NC, NP, C)
    z2, za2 = _tc_mid(p1, z1, s_out, s_in, W2, b2.reshape(1, C))
    q = _sc_gs(za2, dst_p, src_p, zeros_row).reshape(NC, NP, C)
    out = _tc_final(q, z2, s_out, s_in)
    return out[:N]


# speedup vs baseline: 1.1140x; 1.1140x over previous
import functools

import jax
import jax.numpy as jnp
from jax import lax
from jax.experimental import pallas as pl
from jax.experimental.pallas import tpu as pltpu
from jax.experimental.pallas import tpu_sc as plsc

N = 10000
C = 128
E = 320000
NC, NS = 2, 16
NW = NC * NS
K = 128
CH = 79
EW = K * CH
EPAD = NW * EW
NP = CH * 128
NB = NP // 128
ZR = NP // NS
DEGW = 16

_MESH = plsc.VectorSubcoreMesh(
    core_axis_name="c", subcore_axis_name="s", num_cores=NC, num_subcores=NS)


def _sc_deg_body(src_hbm, dst_hbm, ones_hbm, zeros_hbm, cnt_hbm,
                 iv0, iv1, rows, zrow, acc):
    cid = lax.axis_index("c")
    sid = lax.axis_index("s")
    wid = sid * NC + cid
    base = wid * EW
    pltpu.sync_copy(ones_hbm, rows)
    pltpu.sync_copy(zeros_hbm, zrow)
    for j in range(5):
        blk = sid + j * NS
        @pl.when(blk < NB)
        def _():
            pltpu.sync_copy(zrow, acc.at[pl.ds(blk * 128, 128)])

    for phase, idx_hbm in ((0, dst_hbm), (1, src_hbm)):
        plsc.subcore_barrier()
        pltpu.sync_copy(idx_hbm.at[pl.ds(base, K)], iv0)

        def pair(i, carry):
            pltpu.sync_copy(idx_hbm.at[pl.ds(base + (2 * i + 1) * K, K)],
                            iv1)
            pltpu.sync_copy(rows, acc.at[iv0], add=True)
            pltpu.sync_copy(idx_hbm.at[pl.ds(base + (2 * i + 2) * K, K)],
                            iv0)
            pltpu.sync_copy(rows, acc.at[iv1], add=True)
            return carry

        lax.fori_loop(0, (CH - 1) // 2, pair, 0)
        pltpu.sync_copy(rows, acc.at[iv0], add=True)
        plsc.subcore_barrier()
        for j in range(5):
            blk = sid + j * NS
            @pl.when(blk < NB)
            def _():
                pltpu.sync_copy(
                    acc.at[pl.ds(blk * 128, 128)],
                    cnt_hbm.at[pl.ds((phase * NC + cid) * NP + blk * 128,
                                     128)])
                if phase == 0:
                    pltpu.sync_copy(zrow, acc.at[pl.ds(blk * 128, 128)])


_sc_deg = functools.partial(
    pl.kernel,
    out_type=jax.ShapeDtypeStruct((2 * NC * NP, C), jnp.float32),
    mesh=_MESH,
    scratch_types=[
        pltpu.VMEM((K,), jnp.int32),
        pltpu.VMEM((K,), jnp.int32),
        pltpu.VMEM((K, C), jnp.float32),
        pltpu.VMEM((128, C), jnp.float32),
        pltpu.VMEM_SHARED((NP, C), jnp.float32),
    ],
)(_sc_deg_body)

CHT = 2 * CH
CH0 = 129
CH1 = CHT - CH0


def _sc_gs_body(za_hbm, gi_hbm, si_hbm, zeros_hbm, out_hbm,
                giv0, giv1, siv0, siv1, rows0, rows1, acc, gsem0, gsem1):
    cid = lax.axis_index("c")
    sid = lax.axis_index("s")
    base = (sid * CHT + cid * CH0) * K
    nch = jnp.where(cid == 0, CH0, CH1)
    pltpu.sync_copy(zeros_hbm, rows0)
    for j in range(5):
        blk = sid + j * NS
        @pl.when(blk < NB)
        def _():
            pltpu.sync_copy(rows0, acc.at[pl.ds(blk * 128, 128)])
    plsc.subcore_barrier()

    pltpu.sync_copy(gi_hbm.at[pl.ds(base, K)], giv0)
    pltpu.sync_copy(si_hbm.at[pl.ds(base, K)], siv0)
    pltpu.async_copy(za_hbm.at[giv0], rows0, gsem0)

    def pair(i, carry):
        off1 = base + (2 * i + 1) * K
        pltpu.sync_copy(gi_hbm.at[pl.ds(off1, K)], giv1)
        pltpu.sync_copy(si_hbm.at[pl.ds(off1, K)], siv1)
        pltpu.async_copy(za_hbm.at[giv1], rows1, gsem1)
        pltpu.make_async_copy(za_hbm.at[giv0], rows0, gsem0).wait()
        pltpu.sync_copy(rows0, acc.at[siv0], add=True)
        off2 = base + (2 * i + 2) * K
        pltpu.sync_copy(gi_hbm.at[pl.ds(off2, K)], giv0)
        pltpu.sync_copy(si_hbm.at[pl.ds(off2, K)], siv0)
        pltpu.async_copy(za_hbm.at[giv0], rows0, gsem0)
        pltpu.make_async_copy(za_hbm.at[giv1], rows1, gsem1).wait()
        pltpu.sync_copy(rows1, acc.at[siv1], add=True)
        return carry

    lax.fori_loop(0, (nch - 1) // 2, pair, 0)
    pltpu.make_async_copy(za_hbm.at[giv0], rows0, gsem0).wait()
    pltpu.sync_copy(rows0, acc.at[siv0], add=True)
    plsc.subcore_barrier()
    for j in range(5):
        blk = sid + j * NS
        @pl.when(blk < NB)
        def _():
            pltpu.sync_copy(acc.at[pl.ds(blk * 128, 128)],
                            out_hbm.at[pl.ds(cid * NP + blk * 128, 128)])


_sc_gs = functools.partial(
    pl.kernel,
    out_type=jax.ShapeDtypeStruct((NC * NP, C), jnp.float32),
    mesh=_MESH,
    scratch_types=(
        [pltpu.VMEM((K,), jnp.int32) for _ in range(4)]
        + [pltpu.VMEM((K, C), jnp.float32) for _ in range(2)]
        + [pltpu.VMEM_SHARED((NP, C), jnp.float32)]
        + [pltpu.SemaphoreType.DMA for _ in range(2)]
    ),
)(_sc_gs_body)


_DN = (((1,), (1,)), ((), ()))
_RB = 1024
_GRID = (NP + _RB - 1) // _RB


def _tc_first_body(x_ref, w_ref, b_ref, co0, co1, ci0, ci1,
                   z1_ref, za1_ref, so_ref, si_ref):
    z1 = lax.dot_general(
        x_ref[...], w_ref[...], _DN,
        preferred_element_type=jnp.float32) + b_ref[...]
    s_o = lax.rsqrt(co0[...] + co1[...] + 1.0)
    s_i = lax.rsqrt(ci0[...] + ci1[...] + 1.0)
    z1_ref[...] = z1
    za1_ref[...] = s_o * z1
    so_ref[...] = s_o
    si_ref[...] = s_i


def _tc_first(x, w, b, co0, co1, ci0, ci1):
    col = pl.BlockSpec((_RB, 1), lambda i: (i, 0))
    mat = pl.BlockSpec((_RB, C), lambda i: (i, 0))
    return pl.pallas_call(
        _tc_first_body,
        grid=(_GRID,),
        in_specs=[mat,
                  pl.BlockSpec((C, C), lambda i: (0, 0)),
                  pl.BlockSpec((1, C), lambda i: (0, 0)),
                  col, col, col, col],
        out_specs=[mat, mat, col, col],
        out_shape=[jax.ShapeDtypeStruct((NP, C), jnp.float32),
                   jax.ShapeDtypeStruct((NP, C), jnp.float32),
                   jax.ShapeDtypeStruct((NP, 1), jnp.float32),
                   jax.ShapeDtypeStruct((NP, 1), jnp.float32)],
    )(x, w, b, co0, co1, ci0, ci1)


def _tc_mid_body(p_ref, z1_ref, so_ref, si_ref, w_ref, b_ref, z2_ref, za2_ref):
    h = jnp.maximum(
        si_ref[...] * (p_ref[0] + p_ref[1] + so_ref[...] * z1_ref[...]), 0.0)
    z2 = lax.dot_general(
        h, w_ref[...], _DN, preferred_element_type=jnp.float32) + b_ref[...]
    z2_ref[...] = z2
    za2_ref[...] = si_ref[...] * z2


def _tc_mid(p, z1, so, si, w, b):
    col = pl.BlockSpec((_RB, 1), lambda i: (i, 0))
    mat = pl.BlockSpec((_RB, C), lambda i: (i, 0))
    return pl.pallas_call(
        _tc_mid_body,
        grid=(_GRID,),
        in_specs=[pl.BlockSpec((NC, _RB, C), lambda i: (0, i, 0)),
                  mat, col, col,
                  pl.BlockSpec((C, C), lambda i: (0, 0)),
                  pl.BlockSpec((1, C), lambda i: (0, 0))],
        out_specs=[mat, mat],
        out_shape=[jax.ShapeDtypeStruct((NP, C), jnp.float32),
                   jax.ShapeDtypeStruct((NP, C), jnp.float32)],
    )(p, z1, so, si, w, b)


def _tc_final_body(q_ref, z2_ref, so_ref, si_ref, o_ref):
    o_ref[...] = so_ref[...] * (
        q_ref[0] + q_ref[1] + si_ref[...] * z2_ref[...])


def _tc_final(q, z2, so, si):
    col = pl.BlockSpec((_RB, 1), lambda i: (i, 0))
    mat = pl.BlockSpec((_RB, C), lambda i: (i, 0))
    return pl.pallas_call(
        _tc_final_body,
        grid=(_GRID,),
        in_specs=[pl.BlockSpec((NC, _RB, C), lambda i: (0, i, 0)),
                  mat, col, col],
        out_specs=mat,
        out_shape=jax.ShapeDtypeStruct((NP, C), jnp.float32),
    )(q, z2, so, si)


def _sc_testa_body(za_hbm, gi_hbm, out_hbm, giv, rows, sem):
    cid = lax.axis_index("c")
    sid = lax.axis_index("s")
    wid = sid * NC + cid
    base = wid * EW

def kernel(x, edge_index, W1, b1, W2, b2):
    src = edge_index[0].astype(jnp.int32)
    dst = edge_index[1].astype(jnp.int32)
    pad = jnp.full((EPAD - E,), N, jnp.int32)
    src_p = jnp.concatenate([src, pad])
    dst_p = jnp.concatenate([dst, pad])
    x_p = jnp.pad(x, ((0, NP - N), (0, 0)))
    ones_k = jnp.ones((K, C), jnp.float32)
    zeros_row = jnp.zeros((128, C), jnp.float32)

    cb = _sc_deg(src_p, dst_p, ones_k, zeros_row)
    z1, za1, s_out, s_in = _tc_first(
        x_p, W1, b1.reshape(1, C),
        cb[2 * NP:3 * NP, 0:1], cb[3 * NP:, 0:1],
        cb[:NP, 0:1], cb[NP:2 * NP, 0:1])
    p1 = _sc_gs(za1, src_p, dst_p, zeros_row).reshape(NC, NP, C)
    z2, za2 = _tc_mid(p1, z1, s_out, s_in, W2, b2.reshape(1, C))
    q = _sc_gs(za2, dst_p, src_p, zeros_row).reshape(NC, NP, C)
    out = _tc_final(q, z2, s_out, s_in)
    return out[:N]
